# baseline (speedup 1.0000x reference)
import jax
import jax.numpy as jnp
from jax.experimental import pallas as pl
from jax.experimental.pallas import tpu as pltpu

_S_BLK = 256
_NBUF = 4


def _dma_body(s, b, f, table_hbm, out_hbm, tbufs, obufs, in_sems, out_sems):
    n = s // _S_BLK

    def in_copy(i):
        return pltpu.make_async_copy(
            table_hbm.at[pl.ds(i * _S_BLK, _S_BLK), :],
            tbufs.at[i % _NBUF],
            in_sems.at[i % _NBUF],
        )

    def out_copy(i):
        return pltpu.make_async_copy(
            obufs.at[i % _NBUF],
            out_hbm.at[pl.ds(i * _S_BLK, _S_BLK), :, :],
            out_sems.at[i % _NBUF],
        )

    for i in range(min(_NBUF, n)):
        in_copy(i).start()
    for i in range(n):
        if i >= _NBUF:
            out_copy(i - _NBUF).wait()
            in_copy(i).start()
        in_copy(i).wait()
        t = tbufs[i % _NBUF]
        obufs[i % _NBUF] = jnp.broadcast_to(t[:, None, :], (_S_BLK, b, f))
        out_copy(i).start()
    for i in range(max(0, n - _NBUF), n):
        out_copy(i).wait()


def kernel(x, table):
    s, b, f = x.shape
    return pl.pallas_call(
        lambda t, o, tb, ob, isem, osem: _dma_body(s, b, f, t, o, tb, ob, isem, osem),
        in_specs=[pl.BlockSpec(memory_space=pltpu.MemorySpace.HBM)],
        out_specs=pl.BlockSpec(memory_space=pltpu.MemorySpace.HBM),
        out_shape=jax.ShapeDtypeStruct((s, b, f), table.dtype),
        scratch_shapes=[
            pltpu.VMEM((_NBUF, _S_BLK, f), jnp.float32),
            pltpu.VMEM((_NBUF, _S_BLK, b, f), jnp.float32),
            pltpu.SemaphoreType.DMA((_NBUF,)),
            pltpu.SemaphoreType.DMA((_NBUF,)),
        ],
    )(table)


# speedup vs baseline: 2.0234x; 2.0234x over previous
import jax
import jax.numpy as jnp
from jax.experimental import pallas as pl
from jax.experimental.pallas import tpu as pltpu

_S_BLK = 1024
_NBUF = 4


def _dma_body(s, b, f, table_hbm, out_hbm, bufs, in_sems, out_sems):
    n = s // _S_BLK

    def in_copy(i):
        return pltpu.make_async_copy(
            table_hbm.at[pl.ds(i * _S_BLK, _S_BLK), :],
            bufs.at[i % _NBUF],
            in_sems.at[i % _NBUF],
        )

    def out_copy(i, j):
        return pltpu.make_async_copy(
            bufs.at[i % _NBUF],
            out_hbm.at[pl.ds(i * _S_BLK, _S_BLK), j, :],
            out_sems.at[i % _NBUF, j],
        )

    for i in range(min(_NBUF, n)):
        in_copy(i).start()
    for i in range(n):
        if i >= _NBUF:
            for j in range(b):
                out_copy(i - _NBUF, j).wait()
            in_copy(i).start()
        in_copy(i).wait()
        for j in range(b):
            out_copy(i, j).start()
    for i in range(max(0, n - _NBUF), n):
        for j in range(b):
            out_copy(i, j).wait()


def kernel(x, table):
    s, b, f = x.shape
    return pl.pallas_call(
        lambda t, o, bufs, isem, osem: _dma_body(s, b, f, t, o, bufs, isem, osem),
        in_specs=[pl.BlockSpec(memory_space=pltpu.MemorySpace.HBM)],
        out_specs=pl.BlockSpec(memory_space=pltpu.MemorySpace.HBM),
        out_shape=jax.ShapeDtypeStruct((s, b, f), table.dtype),
        scratch_shapes=[
            pltpu.VMEM((_NBUF, _S_BLK, f), jnp.float32),
            pltpu.SemaphoreType.DMA((_NBUF,)),
            pltpu.SemaphoreType.DMA((_NBUF, 4)),
        ],
    )(table)


# speedup vs baseline: 2.0257x; 1.0011x over previous
import jax
import jax.numpy as jnp
from jax.experimental import pallas as pl
from jax.experimental.pallas import tpu as pltpu

_S_BLK = 1024
_NBUF = 8


def _dma_body(s, b, f, table_hbm, out_hbm, bufs, in_sems, out_sems):
    n = s // _S_BLK

    def in_copy(i):
        return pltpu.make_async_copy(
            table_hbm.at[pl.ds(i * _S_BLK, _S_BLK), :],
            bufs.at[i % _NBUF],
            in_sems.at[i % _NBUF],
        )

    def out_copy(i, j):
        return pltpu.make_async_copy(
            bufs.at[i % _NBUF],
            out_hbm.at[pl.ds(i * _S_BLK, _S_BLK), j, :],
            out_sems.at[i % _NBUF, j],
        )

    for i in range(min(_NBUF, n)):
        in_copy(i).start()
    for i in range(n):
        if i >= _NBUF:
            for j in range(b):
                out_copy(i - _NBUF, j).wait()
            in_copy(i).start()
        in_copy(i).wait()
        for j in range(b):
            out_copy(i, j).start()
    for i in range(max(0, n - _NBUF), n):
        for j in range(b):
            out_copy(i, j).wait()


def kernel(x, table):
    s, b, f = x.shape
    return pl.pallas_call(
        lambda t, o, bufs, isem, osem: _dma_body(s, b, f, t, o, bufs, isem, osem),
        in_specs=[pl.BlockSpec(memory_space=pltpu.MemorySpace.HBM)],
        out_specs=pl.BlockSpec(memory_space=pltpu.MemorySpace.HBM),
        out_shape=jax.ShapeDtypeStruct((s, b, f), table.dtype),
        scratch_shapes=[
            pltpu.VMEM((_NBUF, _S_BLK, f), jnp.float32),
            pltpu.SemaphoreType.DMA((_NBUF,)),
            pltpu.SemaphoreType.DMA((_NBUF, 4)),
        ],
    )(table)


# speedup vs baseline: 2.0317x; 1.0030x over previous
import jax
import jax.numpy as jnp
from jax.experimental import pallas as pl
from jax.experimental.pallas import tpu as pltpu

_S_BLK = 1024
_NBUF = 4


def _dma_body(s, b, f, table_hbm, out_hbm, bufs, in_sems, out_sems):
    n = s // _S_BLK

    def in_copy(i):
        return pltpu.make_async_copy(
            table_hbm.at[pl.ds(i * _S_BLK, _S_BLK), :],
            bufs.at[i % _NBUF],
            in_sems.at[i % _NBUF],
        )

    def out_copy(i, j):
        return pltpu.make_async_copy(
            bufs.at[i % _NBUF],
            out_hbm.at[pl.ds(i * _S_BLK, _S_BLK), j, :],
            out_sems.at[i % _NBUF, j],
        )

    for i in range(min(_NBUF, n)):
        in_copy(i).start()
    for i in range(n):
        if i >= _NBUF:
            for j in range(b):
                out_copy(i - _NBUF, j).wait()
            in_copy(i).start()
        in_copy(i).wait()
        for j in range(b):
            out_copy(i, j).start()
    for i in range(max(0, n - _NBUF), n):
        for j in range(b):
            out_copy(i, j).wait()


def kernel(x, table):
    s, b, f = x.shape
    return pl.pallas_call(
        lambda t, o, bufs, isem, osem: _dma_body(s, b, f, t, o, bufs, isem, osem),
        in_specs=[pl.BlockSpec(memory_space=pltpu.MemorySpace.HBM)],
        out_specs=pl.BlockSpec(memory_space=pltpu.MemorySpace.HBM),
        out_shape=jax.ShapeDtypeStruct((s, b, f), table.dtype),
        scratch_shapes=[
            pltpu.VMEM((_NBUF, _S_BLK, f), jnp.float32),
            pltpu.SemaphoreType.DMA((_NBUF,)),
            pltpu.SemaphoreType.DMA((_NBUF, 4)),
        ],
    )(table)
